# Initial kernel scaffold; baseline (speedup 1.0000x reference)
import jax
import jax.numpy as jnp
from jax.experimental import pallas as pl


def kernel(x, edge_index, edge_attr, We1, be1, We2, be2, Wq, bq, Wk, bk, Wv, bv, Wedge, Wskip, bskip, Wbeta):
    raise NotImplementedError("write your pallas kernel here")


# speedup vs baseline: 10.6975x; 10.6975x over previous
import functools
import math

import jax
import jax.numpy as jnp
from jax import lax
from jax.experimental import pallas as pl
from jax.experimental.pallas import tpu as pltpu
from jax.experimental.pallas import tpu_sc as plsc

_F32 = jnp.float32

_N = 10000
_E = 320000
_H = 4
_C = 32
_D = 128

_NC = 2
_NS = 16
_NW = _NC * _NS
_CH = 128
_NCH = 80
_EPW = _CH * _NCH
_EP = _EPW * _NW
_NPAD = 10240
_NPS = _NPAD // _NS

_BN = 400
_BE = 512


def _proj_body(x_ref, w_ref, b_ref, k_ref, v_ref, q_ref, xr_ref):
    y = jnp.dot(x_ref[...], w_ref[...], preferred_element_type=_F32)
    y = y + b_ref[...]
    k_ref[...] = y[:, :_D]
    v_ref[...] = y[:, _D : 2 * _D]
    q_ref[...] = y[:, 2 * _D : 3 * _D]
    xr_ref[...] = y[:, 3 * _D :]


def _proj(x, w, b):
    nspec = pl.BlockSpec((_BN, _D), lambda i: (i, 0))
    nshape = jax.ShapeDtypeStruct((_N, _D), _F32)
    return pl.pallas_call(
        _proj_body,
        grid=(_N // _BN,),
        in_specs=[
            pl.BlockSpec((_BN, _D), lambda i: (i, 0)),
            pl.BlockSpec((_D, 4 * _D), lambda i: (0, 0)),
            pl.BlockSpec((1, 4 * _D), lambda i: (0, 0)),
        ],
        out_specs=[nspec, nspec, nspec, nspec],
        out_shape=[nshape, nshape, nshape, nshape],
    )(x, w, b)


def _sc_gather(kt, vt, qt, src3, dst3):
    mesh = plsc.VectorSubcoreMesh(core_axis_name="c", subcore_axis_name="s")
    eshape = jax.ShapeDtypeStruct((_EP, _D), _F32)

    @functools.partial(
        pl.kernel,
        out_type=[eshape, eshape, eshape],
        mesh=mesh,
        scratch_types=[
            pltpu.VMEM((_NCH, _CH), jnp.int32),
            pltpu.VMEM((_NCH, _CH), jnp.int32),
            pltpu.VMEM((_CH, _D), _F32),
            pltpu.VMEM((_CH, _D), _F32),
            pltpu.VMEM((_CH, _D), _F32),
            pltpu.SemaphoreType.DMA,
            pltpu.SemaphoreType.DMA,
            pltpu.SemaphoreType.DMA,
        ],
    )
    def k(k_hbm, v_hbm, q_hbm, src_hbm, dst_hbm, ks_hbm, vs_hbm, qd_hbm,
          src_v, dst_v, k_buf, v_buf, q_buf, sem1, sem2, sem3):
        wid = lax.axis_index("s") * _NC + lax.axis_index("c")
        base = wid * _EPW
        pltpu.sync_copy(src_hbm.at[wid], src_v)
        pltpu.sync_copy(dst_hbm.at[wid], dst_v)

        @pl.loop(0, _NCH)
        def _(j):
            c1 = pltpu.async_copy(k_hbm.at[src_v.at[j]], k_buf, sem1)
            c2 = pltpu.async_copy(v_hbm.at[src_v.at[j]], v_buf, sem2)
            c3 = pltpu.async_copy(q_hbm.at[dst_v.at[j]], q_buf, sem3)
            c1.wait()
            c2.wait()
            c3.wait()
            pltpu.sync_copy(k_buf, ks_hbm.at[pl.ds(base + j * _CH, _CH)])
            pltpu.sync_copy(v_buf, vs_hbm.at[pl.ds(base + j * _CH, _CH)])
            pltpu.sync_copy(q_buf, qd_hbm.at[pl.ds(base + j * _CH, _CH)])

    return k(kt, vt, qt, src3, dst3)


def _edge_body(ea_ref, ks_ref, vs_ref, qd_ref, we1_ref, be1_ref, we2_ref,
               be2_ref, wed_ref, m_ref, exb_ref):
    h = jnp.dot(ea_ref[...], we1_ref[...], preferred_element_type=_F32)
    h = h + be1_ref[...]
    h = jnp.where(h >= 0, h, 0.15 * h)
    emb = jnp.dot(h, we2_ref[...], preferred_element_type=_F32) + be2_ref[...]
    e = jnp.dot(emb, wed_ref[...], preferred_element_type=_F32)
    prod = qd_ref[...] * (ks_ref[...] + e)
    inv = 1.0 / math.sqrt(float(_C))
    exs = []
    for hh in range(_H):
        a = jnp.sum(prod[:, _C * hh : _C * (hh + 1)], axis=1, keepdims=True)
        exs.append(jnp.exp(a * inv))
    scale = jnp.concatenate(
        [jnp.broadcast_to(exs[hh], (_BE, _C)) for hh in range(_H)], axis=1)
    m_ref[...] = (vs_ref[...] + e) * scale
    exb_ref[...] = scale


def _edge_stage(edge_attr, ks, vs, qd, we1, be1, we2, be2, wed):
    edim = edge_attr.shape[1]
    espec = pl.BlockSpec((_BE, _D), lambda i: (i, 0))
    eshape = jax.ShapeDtypeStruct((_EP, _D), _F32)
    return pl.pallas_call(
        _edge_body,
        grid=(_EP // _BE,),
        in_specs=[
            pl.BlockSpec((_BE, edim), lambda i: (i, 0)),
            espec,
            espec,
            espec,
            pl.BlockSpec((edim, _D), lambda i: (0, 0)),
            pl.BlockSpec((1, _D), lambda i: (0, 0)),
            pl.BlockSpec((_D, _D), lambda i: (0, 0)),
            pl.BlockSpec((1, _D), lambda i: (0, 0)),
            pl.BlockSpec((_D, _D), lambda i: (0, 0)),
        ],
        out_specs=[espec, espec],
        out_shape=[eshape, eshape],
    )(edge_attr, ks, vs, qd, we1, be1, we2, be2, wed)


def _sc_scatter(m, exb, dst3, zeros):
    mesh = plsc.VectorSubcoreMesh(core_axis_name="c", subcore_axis_name="s")
    oshape = jax.ShapeDtypeStruct((_NC, _NPAD, _D), _F32)

    @functools.partial(
        pl.kernel,
        out_type=[oshape, oshape],
        mesh=mesh,
        scratch_types=[
            pltpu.VMEM((_NCH, _CH), jnp.int32),
            pltpu.VMEM((_CH, _D), _F32),
            pltpu.VMEM_SHARED((_NPAD, _D), _F32),
        ],
    )
    def k(m_hbm, exb_hbm, dst_hbm, z_hbm, om_hbm, oe_hbm, idx_v, buf, acc):
        c = lax.axis_index("c")
        s = lax.axis_index("s")
        pltpu.sync_copy(z_hbm.at[pl.ds(s * _NPS, _NPS)],
                        acc.at[pl.ds(s * _NPS, _NPS)])
        pltpu.sync_copy(dst_hbm.at[c, s], idx_v)
        plsc.subcore_barrier()
        base = (c * _NS + s) * _EPW

        @pl.loop(0, _NCH)
        def _(j):
            pltpu.sync_copy(m_hbm.at[pl.ds(base + j * _CH, _CH)], buf)
            pltpu.sync_copy(buf, acc.at[idx_v.at[j]], add=True)

        plsc.subcore_barrier()
        pltpu.sync_copy(acc.at[pl.ds(s * _NPS, _NPS)],
                        om_hbm.at[c, pl.ds(s * _NPS, _NPS)])
        pltpu.sync_copy(z_hbm.at[pl.ds(s * _NPS, _NPS)],
                        acc.at[pl.ds(s * _NPS, _NPS)])
        plsc.subcore_barrier()

        @pl.loop(0, _NCH)
        def _(j):
            pltpu.sync_copy(exb_hbm.at[pl.ds(base + j * _CH, _CH)], buf)
            pltpu.sync_copy(buf, acc.at[idx_v.at[j]], add=True)

        plsc.subcore_barrier()
        pltpu.sync_copy(acc.at[pl.ds(s * _NPS, _NPS)],
                        oe_hbm.at[c, pl.ds(s * _NPS, _NPS)])

    return k(m, exb, dst3, zeros)


def _final_body(m0_ref, m1_ref, e0_ref, e1_ref, xr_ref, u_ref, w_ref,
                out_ref):
    magg = m0_ref[...] + m1_ref[...]
    den = e0_ref[...] + e1_ref[...]
    out = magg / (den + 1e-16)
    xr = xr_ref[...]
    logit = jnp.sum(out * u_ref[...] + xr * w_ref[...], axis=1, keepdims=True)
    beta = jax.nn.sigmoid(logit)
    out_ref[...] = beta * xr + (1.0 - beta) * out


def _final_stage(m0, m1, e0, e1, xr, u, w):
    nspec = pl.BlockSpec((_BN, _D), lambda i: (i, 0))
    return pl.pallas_call(
        _final_body,
        grid=(_N // _BN,),
        in_specs=[
            nspec, nspec, nspec, nspec, nspec,
            pl.BlockSpec((1, _D), lambda i: (0, 0)),
            pl.BlockSpec((1, _D), lambda i: (0, 0)),
        ],
        out_specs=nspec,
        out_shape=jax.ShapeDtypeStruct((_N, _D), _F32),
    )(m0, m1, e0, e1, xr, u, w)


def kernel(x, edge_index, edge_attr, We1, be1, We2, be2, Wq, bq, Wk, bk,
           Wv, bv, Wedge, Wskip, bskip, Wbeta):
    assert x.shape == (_N, _D) and edge_index.shape == (2, _E)
    pad = _EP - _E
    src = jnp.pad(edge_index[0], (0, pad))
    dst = jnp.pad(edge_index[1], (0, pad), constant_values=_N)
    ea_p = jnp.pad(edge_attr, ((0, pad), (0, 0)))

    w_all = jnp.concatenate([Wk, Wv, Wq, Wskip], axis=1)
    b_all = jnp.concatenate([bk, bv, bq, bskip]).reshape(1, 4 * _D)
    kt, vt, qt, xr = _proj(x, w_all, b_all)

    src3 = src.reshape(_NW, _NCH, _CH)
    dstg = dst.reshape(_NW, _NCH, _CH)
    ks, vs, qd = _sc_gather(kt, vt, qt, src3, dstg)

    eargs = jax.lax.optimization_barrier(
        (ea_p, ks, vs, qd, We1, be1.reshape(1, _D), We2,
         be2.reshape(1, _D), Wedge))
    m, exb = jax.lax.optimization_barrier(_edge_stage(*eargs))

    dst3 = dst.reshape(_NC, _NS, _NCH, _CH)
    om, oe = _sc_scatter(m, exb, dst3, jnp.zeros((_NPAD, _D), _F32))

    u = (Wbeta[:_D, 0] + Wbeta[2 * _D :, 0]).reshape(1, _D)
    w = (Wbeta[_D : 2 * _D, 0] - Wbeta[2 * _D :, 0]).reshape(1, _D)
    return _final_stage(om[0, :_N], om[1, :_N], oe[0, :_N], oe[1, :_N],
                        xr, u, w)
